# baseline (speedup 1.0000x reference)
import functools

import jax
import jax.numpy as jnp
from jax import lax
from jax.experimental import pallas as pl
from jax.experimental.pallas import tpu as pltpu
from jax.experimental.pallas import tpu_sc as plsc

B = 16384
S = 50
D = 32
V = 1000000
L = 16
NC, NS = 2, 16
NW = NC * NS
RW = B // NW
GB = 128
SP = 56
KB = RW // GB

VBLK = V // GB
BPW = VBLK // NW
XBLK = VBLK - BPW * NW
CL = 512
NCHUNK = BPW * GB // CL
CVR = V // 4


def _make_cvt_call():
  mesh = plsc.VectorSubcoreMesh(core_axis_name="c", subcore_axis_name="s")

  @functools.partial(
      pl.kernel,
      out_type=jax.ShapeDtypeStruct((CVR, 128), jnp.float32),
      mesh=mesh,
      compiler_params=pltpu.CompilerParams(needs_layout_passes=False,
                                           use_tc_tiling_on_sc=True),
      scratch_types=[
          pltpu.VMEM((D, CL), jnp.float32),
          pltpu.VMEM((CL // 4, 128), jnp.float32),
          pltpu.VMEM((L, 128), jnp.float32),
      ],
  )
  def cvt_kernel(embt_hbm, tail_hbm, cvt_hbm, in_v, out_v, tail_v):
    wid = lax.axis_index("s") * NC + lax.axis_index("c")
    lanes = lax.iota(jnp.int32, L)

    def transpose_chunk(n_ids):
      for g in range(n_ids // L):
        jvec = lanes + g * L
        rowv = jvec >> 2
        colb = (jvec & 3) << 5
        for d in range(D):
          plsc.store_scatter(out_v, [rowv, colb + d], in_v[d, pl.ds(g * L, L)])

    def chunk_body(c, carry):
      lane0 = pl.multiple_of(wid * (BPW * GB) + c * CL, CL)
      pltpu.sync_copy(embt_hbm.at[:, pl.ds(lane0, CL)], in_v)
      transpose_chunk(CL)
      row0 = pl.multiple_of(wid * (BPW * GB // 4) + c * (CL // 4), CL // 4)
      pltpu.sync_copy(out_v, cvt_hbm.at[pl.ds(row0, CL // 4)])
      return carry

    lax.fori_loop(0, NCHUNK, chunk_body, 0)

    @pl.when(wid < XBLK)
    def _():
      lane0 = pl.multiple_of((BPW * NW + wid) * GB, GB)
      pltpu.sync_copy(embt_hbm.at[:, pl.ds(lane0, GB)],
                      in_v.at[:, pl.ds(0, GB)])
      for g in range(GB // L):
        jvec = lanes + g * L
        rowv = jvec >> 2
        colb = (jvec & 3) << 5
        for d in range(D):
          plsc.store_scatter(out_v, [rowv, colb + d], in_v[d, pl.ds(g * L, L)])
      row0 = pl.multiple_of((BPW * NW + wid) * (GB // 4), GB // 4)
      pltpu.sync_copy(out_v.at[pl.ds(0, GB // 4)],
                      cvt_hbm.at[pl.ds(row0, GB // 4)])

    @pl.when(wid == XBLK)
    def _():
      pltpu.sync_copy(tail_hbm, tail_v)
      pltpu.sync_copy(tail_v, cvt_hbm.at[pl.ds(VBLK * GB // 4, L)])

  return cvt_kernel


def _make_sc_call():
  mesh = plsc.VectorSubcoreMesh(core_axis_name="c", subcore_axis_name="s")

  @functools.partial(
      pl.kernel,
      out_type=jax.ShapeDtypeStruct((B, 128), jnp.float32),
      mesh=mesh,
      compiler_params=pltpu.CompilerParams(needs_layout_passes=False,
                                           use_tc_tiling_on_sc=False),
      scratch_types=[
          pltpu.VMEM((RW, SP), jnp.int32),
          pltpu.VMEM((S, KB, GB), jnp.int32),
          pltpu.VMEM((RW, D), jnp.float32),
          pltpu.VMEM((RW,), jnp.float32),
          pltpu.VMEM((RW,), jnp.float32),
          pltpu.VMEM((1, D), jnp.float32),
          pltpu.SemaphoreType.DMA,
      ],
  )
  def sc_kernel(idx_hbm, emb_hbm, out_hbm,
                idxr_v, idxt_v, acc_v, inv_v, nz_v, e0_v, sem):
    wid = lax.axis_index("s") * NC + lax.axis_index("c")
    row0 = pl.multiple_of(wid * RW, RW)

    pltpu.sync_copy(emb_hbm.at[pl.ds(0, 1)], e0_v)
    pltpu.sync_copy(idx_hbm.at[pl.ds(row0, RW), pl.ds(0, SP)], idxr_v)
    lanes = lax.iota(jnp.int32, L)

    for k in range(KB):
      for l in range(GB // L):
        rbase = k * GB + l * L
        idxt_v[0, k, pl.ds(l * L, L)] = plsc.load_gather(
            idxr_v, [lanes + rbase, jnp.zeros((L,), jnp.int32)])
    d0 = [pltpu.async_copy(emb_hbm.at[idxt_v.at[0, k]],
                           acc_v.at[pl.ds(k * GB, GB)], sem)
          for k in range(KB)]

    for k in range(KB):
      for l in range(GB // L):
        gbase = k * GB + l * L
        rvec = lanes + gbase

        def body(s, cnt, k=k, l=l, rvec=rvec):
          vals = plsc.load_gather(idxr_v, [rvec, jnp.full((L,), s, jnp.int32)])
          idxt_v[s, k, pl.ds(l * L, L)] = vals
          return cnt + (vals != 0).astype(jnp.int32)

        cnt0 = (idxt_v[0, k, pl.ds(l * L, L)] != 0).astype(jnp.int32)
        cnt = lax.fori_loop(1, S, body, cnt0)
        cntf = cnt.astype(jnp.float32)
        inv_v[pl.ds(gbase, L)] = jnp.where(cnt == 0, 0.0,
                                           1.0 / (cntf + 1e-8))
        nz_v[pl.ds(gbase, L)] = jnp.float32(S) - cntf

    for dd in d0:
      dd.wait()

    def fire(s, carry):
      for k in range(KB):
        pltpu.async_copy(emb_hbm.at[idxt_v.at[s, k]],
                         acc_v.at[pl.ds(k * GB, GB)], sem, add=True)
      return carry

    lax.fori_loop(1, S, fire, 0)

    def drain(i, carry):
      pltpu.make_async_copy(emb_hbm.at[idxt_v.at[0, 0]],
                            acc_v.at[pl.ds(0, GB)], sem).wait()
      return carry

    lax.fori_loop(0, (S - 1) * KB, drain, 0)

    e00 = e0_v[0, 0:L]
    e01 = e0_v[0, L:D]

    def row_body(r, carry):
      isplat = jnp.full((L,), r, jnp.int32)
      nz = plsc.load_gather(nz_v, [isplat])
      inv = plsc.load_gather(inv_v, [isplat])
      acc_v[r, 0:L] = (acc_v[r, 0:L] - nz * e00) * inv
      acc_v[r, L:D] = (acc_v[r, L:D] - nz * e01) * inv
      return carry

    lax.fori_loop(0, RW, row_body, 0)

    pltpu.sync_copy(acc_v, out_hbm.at[pl.ds(row0, RW), pl.ds(0, D)])

  return sc_kernel


_make_cvt_call = functools.cache(_make_cvt_call)
_make_sc_call = functools.cache(_make_sc_call)


def kernel(indices, embeddings):
  tail = embeddings[VBLK * GB:].reshape(L, 128)
  cvt = _make_cvt_call()(embeddings.T, tail)
  emb_rows = cvt.reshape(V, D)
  idx_pad = jnp.pad(indices.astype(jnp.int32), ((0, 0), (0, 128 - S)))
  padded = _make_sc_call()(idx_pad, emb_rows)
  return padded[:, :D]


# speedup vs baseline: 1.1442x; 1.1442x over previous
import functools

import jax
import jax.numpy as jnp
from jax import lax
from jax.experimental import pallas as pl
from jax.experimental.pallas import tpu as pltpu
from jax.experimental.pallas import tpu_sc as plsc

B = 16384
S = 50
D = 32
V = 1000000
L = 16
NC, NS = 2, 16
NW = NC * NS
RW = B // NW
GB = 128
SP = 56
KB = RW // GB

VBLK = V // GB
BPW = VBLK // NW
XBLK = VBLK - BPW * NW
CL = 512
NCHUNK = BPW * GB // CL
CVR = V // 4


def _make_cvt_call():
  mesh = plsc.VectorSubcoreMesh(core_axis_name="c", subcore_axis_name="s")

  @functools.partial(
      pl.kernel,
      out_type=jax.ShapeDtypeStruct((CVR, 128), jnp.float32),
      mesh=mesh,
      compiler_params=pltpu.CompilerParams(needs_layout_passes=False,
                                           use_tc_tiling_on_sc=True),
      scratch_types=[
          pltpu.VMEM((2, D, CL), jnp.float32),
          pltpu.VMEM((2, CL // 4, 128), jnp.float32),
          pltpu.VMEM((L, 128), jnp.float32),
          pltpu.SemaphoreType.DMA,
          pltpu.SemaphoreType.DMA,
      ],
  )
  def cvt_kernel(embt_hbm, tail_hbm, cvt_hbm, in_v, out_v, tail_v,
                 sem_i, sem_o):
    wid = lax.axis_index("s") * NC + lax.axis_index("c")
    lanes = lax.iota(jnp.int32, L)

    def in_slice(c):
      lane0 = pl.multiple_of(wid * (BPW * GB) + c * CL, CL)
      return embt_hbm.at[:, pl.ds(lane0, CL)]

    def out_slice(c):
      row0 = pl.multiple_of(wid * (BPW * GB // 4) + c * (CL // 4), CL // 4)
      return cvt_hbm.at[pl.ds(row0, CL // 4)]

    def fire_in(c, p):
      pltpu.async_copy(in_slice(c), in_v.at[p], sem_i)

    def wait_in(p):
      pltpu.make_async_copy(in_slice(0), in_v.at[p], sem_i).wait()

    def wait_out(p):
      pltpu.make_async_copy(in_slice(0), out_v.at[p], sem_o).wait()

    def transpose_chunk(p, n_ids):
      for g in range(n_ids // L):
        jvec = lanes + g * L
        rowv = jvec >> 2
        colb = (jvec & 3) << 5
        for d in range(D):
          plsc.store_scatter(out_v.at[p], [rowv, colb + d],
                             in_v[p, d, pl.ds(g * L, L)])

    fire_in(0, 0)

    def pair_body(q, carry):
      c0 = q * 2
      wait_in(0)
      fire_in(c0 + 1, 1)

      @pl.when(q >= 1)
      def _():
        wait_out(0)
        wait_out(1)

      transpose_chunk(0, CL)
      pltpu.async_copy(out_v.at[0], out_slice(c0), sem_o)
      wait_in(1)

      @pl.when(q <= NCHUNK // 2 - 2)
      def _():
        fire_in(c0 + 2, 0)

      transpose_chunk(1, CL)
      pltpu.async_copy(out_v.at[1], out_slice(c0 + 1), sem_o)
      return carry

    lax.fori_loop(0, NCHUNK // 2, pair_body, 0)

    pltpu.sync_copy(in_slice(NCHUNK - 1), in_v.at[0])
    wait_out(0)
    wait_out(1)
    transpose_chunk(0, CL)
    pltpu.sync_copy(out_v.at[0], out_slice(NCHUNK - 1))

    @pl.when(wid < XBLK)
    def _():
      lane0 = pl.multiple_of((BPW * NW + wid) * GB, GB)
      pltpu.sync_copy(embt_hbm.at[:, pl.ds(lane0, GB)],
                      in_v.at[0, :, pl.ds(0, GB)])
      for g in range(GB // L):
        jvec = lanes + g * L
        rowv = jvec >> 2
        colb = (jvec & 3) << 5
        for d in range(D):
          plsc.store_scatter(out_v.at[0], [rowv, colb + d],
                             in_v[0, d, pl.ds(g * L, L)])
      row0 = pl.multiple_of((BPW * NW + wid) * (GB // 4), GB // 4)
      pltpu.sync_copy(out_v.at[0, pl.ds(0, GB // 4)],
                      cvt_hbm.at[pl.ds(row0, GB // 4)])

    @pl.when(wid == XBLK)
    def _():
      pltpu.sync_copy(tail_hbm, tail_v)
      pltpu.sync_copy(tail_v, cvt_hbm.at[pl.ds(VBLK * GB // 4, L)])

  return cvt_kernel


def _make_sc_call():
  mesh = plsc.VectorSubcoreMesh(core_axis_name="c", subcore_axis_name="s")

  @functools.partial(
      pl.kernel,
      out_type=jax.ShapeDtypeStruct((B, 128), jnp.float32),
      mesh=mesh,
      compiler_params=pltpu.CompilerParams(needs_layout_passes=False,
                                           use_tc_tiling_on_sc=False),
      scratch_types=[
          pltpu.VMEM((RW, SP), jnp.int32),
          pltpu.VMEM((S, KB, GB), jnp.int32),
          pltpu.VMEM((RW, D), jnp.float32),
          pltpu.VMEM((RW,), jnp.float32),
          pltpu.VMEM((RW,), jnp.float32),
          pltpu.VMEM((1, D), jnp.float32),
          pltpu.SemaphoreType.DMA,
      ],
  )
  def sc_kernel(idx_hbm, emb_hbm, out_hbm,
                idxr_v, idxt_v, acc_v, inv_v, nz_v, e0_v, sem):
    wid = lax.axis_index("s") * NC + lax.axis_index("c")
    row0 = pl.multiple_of(wid * RW, RW)

    pltpu.sync_copy(emb_hbm.at[pl.ds(0, 1)], e0_v)
    pltpu.sync_copy(idx_hbm.at[pl.ds(row0, RW), pl.ds(0, SP)], idxr_v)
    lanes = lax.iota(jnp.int32, L)

    for k in range(KB):
      for l in range(GB // L):
        rbase = k * GB + l * L
        idxt_v[0, k, pl.ds(l * L, L)] = plsc.load_gather(
            idxr_v, [lanes + rbase, jnp.zeros((L,), jnp.int32)])
    d0 = [pltpu.async_copy(emb_hbm.at[idxt_v.at[0, k]],
                           acc_v.at[pl.ds(k * GB, GB)], sem)
          for k in range(KB)]

    for k in range(KB):
      for l in range(GB // L):
        gbase = k * GB + l * L
        rvec = lanes + gbase

        def body(s, cnt, k=k, l=l, rvec=rvec):
          vals = plsc.load_gather(idxr_v, [rvec, jnp.full((L,), s, jnp.int32)])
          idxt_v[s, k, pl.ds(l * L, L)] = vals
          return cnt + (vals != 0).astype(jnp.int32)

        cnt0 = (idxt_v[0, k, pl.ds(l * L, L)] != 0).astype(jnp.int32)
        cnt = lax.fori_loop(1, S, body, cnt0)
        cntf = cnt.astype(jnp.float32)
        inv_v[pl.ds(gbase, L)] = jnp.where(cnt == 0, 0.0,
                                           1.0 / (cntf + 1e-8))
        nz_v[pl.ds(gbase, L)] = jnp.float32(S) - cntf

    for dd in d0:
      dd.wait()

    def fire(s, carry):
      for k in range(KB):
        pltpu.async_copy(emb_hbm.at[idxt_v.at[s, k]],
                         acc_v.at[pl.ds(k * GB, GB)], sem, add=True)
      return carry

    lax.fori_loop(1, S, fire, 0)

    def drain(i, carry):
      pltpu.make_async_copy(emb_hbm.at[idxt_v.at[0, 0]],
                            acc_v.at[pl.ds(0, GB)], sem).wait()
      return carry

    lax.fori_loop(0, (S - 1) * KB, drain, 0)

    e00 = e0_v[0, 0:L]
    e01 = e0_v[0, L:D]

    def row_body(r, carry):
      isplat = jnp.full((L,), r, jnp.int32)
      nz = plsc.load_gather(nz_v, [isplat])
      inv = plsc.load_gather(inv_v, [isplat])
      acc_v[r, 0:L] = (acc_v[r, 0:L] - nz * e00) * inv
      acc_v[r, L:D] = (acc_v[r, L:D] - nz * e01) * inv
      return carry

    lax.fori_loop(0, RW, row_body, 0)

    pltpu.sync_copy(acc_v, out_hbm.at[pl.ds(row0, RW), pl.ds(0, D)])

  return sc_kernel


_make_cvt_call = functools.cache(_make_cvt_call)
_make_sc_call = functools.cache(_make_sc_call)


def kernel(indices, embeddings):
  tail = embeddings[VBLK * GB:].reshape(L, 128)
  cvt = _make_cvt_call()(embeddings.T, tail)
  emb_rows = cvt.reshape(V, D)
  idx_pad = jnp.pad(indices.astype(jnp.int32), ((0, 0), (0, 128 - S)))
  padded = _make_sc_call()(idx_pad, emb_rows)
  return padded[:, :D]


# speedup vs baseline: 1.4667x; 1.2818x over previous
import functools

import jax
import jax.numpy as jnp
from jax import lax
from jax.experimental import pallas as pl
from jax.experimental.pallas import tpu as pltpu
from jax.experimental.pallas import tpu_sc as plsc

B = 16384
S = 50
D = 32
L = 16
NC, NS = 2, 16
NW = NC * NS
RW = B // NW
GB = 128
SP = 56
KB = RW // GB


def _make_sc_call():
  mesh = plsc.VectorSubcoreMesh(core_axis_name="c", subcore_axis_name="s")

  @functools.partial(
      pl.kernel,
      out_type=jax.ShapeDtypeStruct((B, 128), jnp.float32),
      mesh=mesh,
      compiler_params=pltpu.CompilerParams(needs_layout_passes=False,
                                           use_tc_tiling_on_sc=False),
      scratch_types=[
          pltpu.VMEM((RW, SP), jnp.int32),
          pltpu.VMEM((S, KB, GB), jnp.int32),
          pltpu.VMEM((RW, D), jnp.float32),
          pltpu.VMEM((RW,), jnp.float32),
          pltpu.VMEM((RW,), jnp.float32),
          pltpu.VMEM((1, D), jnp.float32),
          pltpu.SemaphoreType.DMA,
      ],
  )
  def sc_kernel(idx_hbm, emb_hbm, out_hbm,
                idxr_v, idxt_v, acc_v, inv_v, nz_v, e0_v, sem):
    wid = lax.axis_index("s") * NC + lax.axis_index("c")
    row0 = pl.multiple_of(wid * RW, RW)

    pltpu.sync_copy(emb_hbm.at[pl.ds(0, 1)], e0_v)
    pltpu.sync_copy(idx_hbm.at[pl.ds(row0, RW), pl.ds(0, SP)], idxr_v)
    lanes = lax.iota(jnp.int32, L)

    for k in range(KB):
      for l in range(GB // L):
        rbase = k * GB + l * L
        idxt_v[0, k, pl.ds(l * L, L)] = plsc.load_gather(
            idxr_v, [lanes + rbase, jnp.zeros((L,), jnp.int32)])
    d0 = [pltpu.async_copy(emb_hbm.at[idxt_v.at[0, k]],
                           acc_v.at[pl.ds(k * GB, GB)], sem)
          for k in range(KB)]

    for k in range(KB):
      for l in range(GB // L):
        gbase = k * GB + l * L
        rvec = lanes + gbase

        def body(s, cnt, k=k, l=l, rvec=rvec):
          vals = plsc.load_gather(idxr_v, [rvec, jnp.full((L,), s, jnp.int32)])
          idxt_v[s, k, pl.ds(l * L, L)] = vals
          return cnt + (vals != 0).astype(jnp.int32)

        cnt0 = (idxt_v[0, k, pl.ds(l * L, L)] != 0).astype(jnp.int32)
        cnt = lax.fori_loop(1, S, body, cnt0)
        cntf = cnt.astype(jnp.float32)
        inv_v[pl.ds(gbase, L)] = jnp.where(cnt == 0, 0.0,
                                           1.0 / (cntf + 1e-8))
        nz_v[pl.ds(gbase, L)] = jnp.float32(S) - cntf

    for dd in d0:
      dd.wait()

    def fire(s, carry):
      for k in range(KB):
        pltpu.async_copy(emb_hbm.at[idxt_v.at[s, k]],
                         acc_v.at[pl.ds(k * GB, GB)], sem, add=True)
      return carry

    lax.fori_loop(1, S, fire, 0)

    def drain(i, carry):
      pltpu.make_async_copy(emb_hbm.at[idxt_v.at[0, 0]],
                            acc_v.at[pl.ds(0, GB)], sem).wait()
      return carry

    lax.fori_loop(0, (S - 1) * KB, drain, 0)

    e00 = e0_v[0, 0:L]
    e01 = e0_v[0, L:D]

    def row_body(r, carry):
      isplat = jnp.full((L,), r, jnp.int32)
      nz = plsc.load_gather(nz_v, [isplat])
      inv = plsc.load_gather(inv_v, [isplat])
      acc_v[r, 0:L] = (acc_v[r, 0:L] - nz * e00) * inv
      acc_v[r, L:D] = (acc_v[r, L:D] - nz * e01) * inv
      return carry

    lax.fori_loop(0, RW, row_body, 0)

    pltpu.sync_copy(acc_v, out_hbm.at[pl.ds(row0, RW), pl.ds(0, D)])

  return sc_kernel


_make_sc_call = functools.cache(_make_sc_call)


def kernel(indices, embeddings):
  idx_pad = jnp.pad(indices.astype(jnp.int32), ((0, 0), (0, 128 - S)))
  padded = _make_sc_call()(idx_pad, embeddings)
  return padded[:, :D]
